# Initial kernel scaffold; baseline (speedup 1.0000x reference)
import jax
import jax.numpy as jnp
from jax.experimental import pallas as pl


def kernel(batch, user_table, item_table):
    raise NotImplementedError("write your pallas kernel here")


# speedup vs baseline: 7.5751x; 7.5751x over previous
import functools

import jax
import jax.numpy as jnp
from jax import lax
from jax.experimental import pallas as pl
from jax.experimental.pallas import tpu as pltpu
from jax.experimental.pallas import tpu_sc as plsc

_B, _L, _N, _D = 1024, 20, 50, 64
_NC, _NS, _LANES = 2, 16, 16
_NW = _NC * _NS
_PAIRS = _B * _L
_PPW = _PAIRS // _NW
_CH = 16
_NCHUNK = _PPW // _CH
_CITEMS = _CH * _N


def _sc_body(user_tab, item_tab, uidx_hbm, iidx_hbm, out_hbm,
             uidx_v, iidx_v, urows, irows, part, outb, sem):
    wid = lax.axis_index("s") * _NC + lax.axis_index("c")
    base_pair = wid * _PPW

    def chunk_body(c, carry):
        p0 = base_pair + c * _CH
        pltpu.sync_copy(uidx_hbm.at[pl.ds(p0, _CH)], uidx_v)
        pltpu.sync_copy(iidx_hbm.at[pl.ds(p0, _CH)], iidx_v)
        copies = [pltpu.async_copy(user_tab.at[uidx_v], urows, sem)]
        for p in range(_CH):
            copies.append(
                pltpu.async_copy(item_tab.at[iidx_v.at[p]],
                                 irows.at[pl.ds(p * _N, _N)], sem))
        for cp in copies:
            cp.wait()

        def pair_body(p, carry2):
            u = [urows[p, pl.ds(k * _LANES, _LANES)] for k in range(4)]

            def item_body(j, carry3):
                r = p * _N + j
                acc = irows[r, pl.ds(0, _LANES)] * u[0]
                for k in range(1, 4):
                    acc = acc + irows[r, pl.ds(k * _LANES, _LANES)] * u[k]
                part[pl.ds(r * _LANES, _LANES)] = acc
                return carry3

            return lax.fori_loop(0, _N, item_body, carry2, unroll=2)

        lax.fori_loop(0, _CH, pair_body, 0)

        iv = lax.iota(jnp.int32, _LANES) * _LANES

        def group_body(g, carry2):
            base = g * (_LANES * _LANES)
            acc = plsc.load_gather(part, [base + iv])
            for k in range(1, _LANES):
                acc = acc + plsc.load_gather(part, [base + iv + k])
            outb[pl.ds(g * _LANES, _LANES)] = acc
            return carry2

        lax.fori_loop(0, _CITEMS // _LANES, group_body, 0)
        pltpu.sync_copy(outb, out_hbm.at[pl.ds(p0 * _N, _CITEMS)])
        return carry

    lax.fori_loop(0, _NCHUNK, chunk_body, 0)


@jax.jit
def _sc_call(user_table, item_table, uidx, iidx):
    mesh = plsc.VectorSubcoreMesh(
        core_axis_name="c", subcore_axis_name="s",
        num_cores=_NC, num_subcores=_NS)
    return pl.kernel(
        _sc_body,
        out_type=jax.ShapeDtypeStruct((_PAIRS * _N,), jnp.float32),
        mesh=mesh,
        scratch_types=[
            pltpu.VMEM((_CH,), jnp.int32),
            pltpu.VMEM((_CH, _N), jnp.int32),
            pltpu.VMEM((_CH, _D), jnp.float32),
            pltpu.VMEM((_CITEMS, _D), jnp.float32),
            pltpu.VMEM((_CITEMS * _LANES,), jnp.float32),
            pltpu.VMEM((_CITEMS,), jnp.float32),
            pltpu.SemaphoreType.DMA,
        ],
        compiler_params=pltpu.CompilerParams(
            needs_layout_passes=False, use_tc_tiling_on_sc=False),
    )(user_table, item_table, uidx, iidx)


def kernel(batch, user_table, item_table):
    batch = batch.astype(jnp.int32)
    uidx = batch[:, :, 0].reshape(_PAIRS)
    iidx = batch[:, :, 1:].reshape(_PAIRS, _N)
    flat = _sc_call(user_table, item_table, uidx, iidx)
    return flat.reshape(_B, _L, _N)


# speedup vs baseline: 9.0066x; 1.1890x over previous
import functools

import jax
import jax.numpy as jnp
from jax import lax
from jax.experimental import pallas as pl
from jax.experimental.pallas import tpu as pltpu
from jax.experimental.pallas import tpu_sc as plsc

_B, _L, _N, _D = 1024, 20, 50, 64
_NC, _NS, _LANES = 2, 16, 16
_NW = _NC * _NS
_PAIRS = _B * _L
_PPW = _PAIRS // _NW
_CH = 16
_NCHUNK = _PPW // _CH
_CITEMS = _CH * _N


def _sc_body(user_tab, item_tab, uidx_hbm, iidx_hbm, out_hbm,
             uidx0, uidx1, iidx0, iidx1, urows0, urows1, irows0, irows1,
             outb0, outb1, part, sem0, sem1):
    bufs = ((uidx0, iidx0, urows0, irows0, outb0, sem0),
            (uidx1, iidx1, urows1, irows1, outb1, sem1))
    wid = lax.axis_index("s") * _NC + lax.axis_index("c")
    base_pair = wid * _PPW

    def start(c, b):
        uidx_v, iidx_v, urows, irows, _, sem = bufs[b]
        p0 = base_pair + c * _CH
        pltpu.sync_copy(uidx_hbm.at[pl.ds(p0, _CH)], uidx_v)
        pltpu.sync_copy(iidx_hbm.at[pl.ds(p0, _CH)], iidx_v)
        pltpu.async_copy(user_tab.at[uidx_v], urows, sem)
        for p in range(_CH):
            pltpu.async_copy(item_tab.at[iidx_v.at[p]],
                             irows.at[pl.ds(p * _N, _N)], sem)

    def drain(b):
        uidx_v, iidx_v, urows, irows, _, sem = bufs[b]
        pltpu.make_async_copy(user_tab.at[uidx_v], urows, sem).wait()
        for p in range(_CH):
            pltpu.make_async_copy(item_tab.at[iidx_v.at[p]],
                                  irows.at[pl.ds(p * _N, _N)], sem).wait()

    def compute(c, b):
        _, _, urows, irows, outb, _ = bufs[b]
        p0 = base_pair + c * _CH

        def pair_body(p, carry2):
            u = [urows[p, pl.ds(k * _LANES, _LANES)] for k in range(4)]

            def item_body(j, carry3):
                r = p * _N + j
                acc = irows[r, pl.ds(0, _LANES)] * u[0]
                for k in range(1, 4):
                    acc = acc + irows[r, pl.ds(k * _LANES, _LANES)] * u[k]
                part[pl.ds(r * _LANES, _LANES)] = acc
                return carry3

            return lax.fori_loop(0, _N, item_body, carry2, unroll=5)

        lax.fori_loop(0, _CH, pair_body, 0)

        iv = lax.iota(jnp.int32, _LANES) * _LANES

        def group_body(g, carry2):
            base = g * (_LANES * _LANES)
            acc = plsc.load_gather(part, [base + iv])
            for k in range(1, _LANES):
                acc = acc + plsc.load_gather(part, [base + iv + k])
            outb[pl.ds(g * _LANES, _LANES)] = acc
            return carry2

        lax.fori_loop(0, _CITEMS // _LANES, group_body, 0, unroll=2)
        pltpu.sync_copy(outb, out_hbm.at[pl.ds(p0 * _N, _CITEMS)])

    start(0, 0)
    start(1, 1)

    def chunk2_body(c2, carry):
        for b in range(2):
            c = c2 * 2 + b
            drain(b)
            compute(c, b)

            @pl.when(c2 < _NCHUNK // 2 - 1)
            def _():
                start(c + 2, b)

        return carry

    lax.fori_loop(0, _NCHUNK // 2, chunk2_body, 0)


@jax.jit
def _sc_call(user_table, item_table, uidx, iidx):
    mesh = plsc.VectorSubcoreMesh(
        core_axis_name="c", subcore_axis_name="s",
        num_cores=_NC, num_subcores=_NS)
    dbl = lambda t: [t, t]
    return pl.kernel(
        _sc_body,
        out_type=jax.ShapeDtypeStruct((_PAIRS * _N,), jnp.float32),
        mesh=mesh,
        scratch_types=(
            dbl(pltpu.VMEM((_CH,), jnp.int32))
            + dbl(pltpu.VMEM((_CH, _N), jnp.int32))
            + dbl(pltpu.VMEM((_CH, _D), jnp.float32))
            + dbl(pltpu.VMEM((_CITEMS, _D), jnp.float32))
            + dbl(pltpu.VMEM((_CITEMS,), jnp.float32))
            + [pltpu.VMEM((_CITEMS * _LANES,), jnp.float32),
               pltpu.SemaphoreType.DMA, pltpu.SemaphoreType.DMA]
        ),
        compiler_params=pltpu.CompilerParams(
            needs_layout_passes=False, use_tc_tiling_on_sc=False),
    )(user_table, item_table, uidx, iidx)


def kernel(batch, user_table, item_table):
    batch = batch.astype(jnp.int32)
    uidx = batch[:, :, 0].reshape(_PAIRS)
    iidx = batch[:, :, 1:].reshape(_PAIRS, _N)
    flat = _sc_call(user_table, item_table, uidx, iidx)
    return flat.reshape(_B, _L, _N)


# speedup vs baseline: 14.5837x; 1.6192x over previous
import functools

import jax
import jax.numpy as jnp
from jax import lax
from jax.experimental import pallas as pl
from jax.experimental.pallas import tpu as pltpu
from jax.experimental.pallas import tpu_sc as plsc

_B, _L, _N, _D = 1024, 20, 50, 64
_NC, _NS, _LANES = 2, 16, 16
_NW = _NC * _NS
_PAIRS = _B * _L
_PPW = _PAIRS // _NW
_CH = 16
_NCHUNK = _PPW // _CH
_CITEMS = _CH * _N


def _sc_body(user_tab, item_tab, uidx_hbm, iidx_hbm, out_hbm,
             uidx0, uidx1, iidx0, iidx1, urows0, urows1, irows0, irows1,
             outb0, outb1, sem0, sem1):
    bufs = ((uidx0, iidx0, urows0, irows0, outb0, sem0),
            (uidx1, iidx1, urows1, irows1, outb1, sem1))
    wid = lax.axis_index("s") * _NC + lax.axis_index("c")
    base_pair = wid * _PPW

    iota = lax.iota(jnp.int32, _LANES)
    lo8 = iota < 8
    folds = {w: jnp.bitwise_xor(iota, w // 2) for w in (16, 8, 4, 2)}
    packs = {w: (jnp.remainder(iota, 8) // (w // 2)) * w
                + jnp.remainder(iota, w // 2)
             for w in (16, 8, 4, 2)}

    def take(v, idx):
        return jnp.take_along_axis(v, idx, axis=0, mode="promise_in_bounds")

    def combine(a, b, w):
        fa = a + take(a, folds[w])
        fb = b + take(b, folds[w])
        return jnp.where(lo8, take(fa, packs[w]), take(fb, packs[w]))

    def tree_push(stack, acc):
        lvl, vec = 0, acc
        while stack and stack[-1][0] == lvl:
            _, prev = stack.pop()
            vec = combine(prev, vec, 16 >> lvl)
            lvl += 1
        stack.append((lvl, vec))

    def partial(irows, r, u):
        acc = irows[r, pl.ds(0, _LANES)] * u[0]
        for k in range(1, 4):
            acc = acc + irows[r, pl.ds(k * _LANES, _LANES)] * u[k]
        return acc

    def start(c, b):
        uidx_v, iidx_v, urows, irows, _, sem = bufs[b]
        p0 = base_pair + c * _CH
        pltpu.sync_copy(uidx_hbm.at[pl.ds(p0, _CH)], uidx_v)
        pltpu.sync_copy(iidx_hbm.at[pl.ds(p0, _CH)], iidx_v)
        pltpu.async_copy(user_tab.at[uidx_v], urows, sem)
        for p in range(_CH):
            pltpu.async_copy(item_tab.at[iidx_v.at[p]],
                             irows.at[pl.ds(p * _N, _N)], sem)

    def drain(b):
        uidx_v, iidx_v, urows, irows, _, sem = bufs[b]
        pltpu.make_async_copy(user_tab.at[uidx_v], urows, sem).wait()
        for p in range(_CH):
            pltpu.make_async_copy(item_tab.at[iidx_v.at[p]],
                                  irows.at[pl.ds(p * _N, _N)], sem).wait()

    def compute(c, b):
        _, _, urows, irows, outb, _ = bufs[b]
        p0 = base_pair + c * _CH

        def pair_body(p, carry2):
            u = [urows[p, pl.ds(k * _LANES, _LANES)] for k in range(4)]
            for blk in range(3):
                stack = []
                for j in range(_LANES):
                    tree_push(stack, partial(irows, p * _N + blk * 16 + j, u))
                outb[pl.ds(p * _N + blk * 16, _LANES)] = stack[0][1]
            return carry2

        lax.fori_loop(0, _CH, pair_body, 0)

        def tail_body(g, carry2):
            stack = []
            for ps in range(8):
                u = [urows[g * 8 + ps, pl.ds(k * _LANES, _LANES)]
                     for k in range(4)]
                r = (g * 8 + ps) * _N + 48
                tree_push(stack, partial(irows, r, u))
                tree_push(stack, partial(irows, r + 1, u))
            idx = (g * 8 + iota // 2) * _N + 48 + jnp.remainder(iota, 2)
            plsc.store_scatter(outb, [idx], stack[0][1])
            return carry2

        lax.fori_loop(0, 2, tail_body, 0)
        pltpu.sync_copy(outb, out_hbm.at[pl.ds(p0 * _N, _CITEMS)])

    start(0, 0)
    start(1, 1)

    def chunk2_body(c2, carry):
        for b in range(2):
            c = c2 * 2 + b
            drain(b)
            compute(c, b)

            @pl.when(c2 < _NCHUNK // 2 - 1)
            def _():
                start(c + 2, b)

        return carry

    lax.fori_loop(0, _NCHUNK // 2, chunk2_body, 0)


@jax.jit
def _sc_call(user_table, item_table, uidx, iidx):
    mesh = plsc.VectorSubcoreMesh(
        core_axis_name="c", subcore_axis_name="s",
        num_cores=_NC, num_subcores=_NS)
    dbl = lambda t: [t, t]
    return pl.kernel(
        _sc_body,
        out_type=jax.ShapeDtypeStruct((_PAIRS * _N,), jnp.float32),
        mesh=mesh,
        scratch_types=(
            dbl(pltpu.VMEM((_CH,), jnp.int32))
            + dbl(pltpu.VMEM((_CH, _N), jnp.int32))
            + dbl(pltpu.VMEM((_CH, _D), jnp.float32))
            + dbl(pltpu.VMEM((_CITEMS, _D), jnp.float32))
            + dbl(pltpu.VMEM((_CITEMS,), jnp.float32))
            + [pltpu.SemaphoreType.DMA, pltpu.SemaphoreType.DMA]
        ),
        compiler_params=pltpu.CompilerParams(
            needs_layout_passes=False, use_tc_tiling_on_sc=False),
    )(user_table, item_table, uidx, iidx)


def kernel(batch, user_table, item_table):
    batch = batch.astype(jnp.int32)
    uidx = batch[:, :, 0].reshape(_PAIRS)
    iidx = batch[:, :, 1:].reshape(_PAIRS, _N)
    flat = _sc_call(user_table, item_table, uidx, iidx)
    return flat.reshape(_B, _L, _N)
